# baseline (speedup 1.0000x reference)
import functools

import jax
import jax.numpy as jnp
from jax import lax
from jax.experimental import pallas as pl
from jax.experimental.pallas import tpu as pltpu
from jax.experimental.pallas import tpu_sc as plsc

N = 4096
D = 512
KVD = 1024
T = 6
TP = 8
NH = 8
HD = 64
SCALE = float(HD) ** -0.5
SW = 128
F32 = jnp.float32
I32 = jnp.int32


def _k12_body(x_ref, w1_ref, b1_ref, wh_ref, bh_ref, wt_ref, bt_ref,
              wkv_ref, bkv_ref,
              h_ref, s_ref, eh_ref, et_ref, kk_ref, vv_ref, hv):
    p = pl.program_id(0)
    j = pl.program_id(1)
    r = x_ref.shape[0]

    @pl.when(p == 0)
    def _():
        h = (jnp.dot(x_ref[...], w1_ref[...], preferred_element_type=F32)
             + b1_ref[...])
        h = jnp.where(h >= 0, h, 0.01 * h)
        h_ref[...] = h
        hv[pl.ds(j * r, r), :] = h

        @pl.when(j == 0)
        def _():
            s_ref[...] = jnp.zeros_like(s_ref)

        s_ref[...] += jnp.sum(h, axis=0, keepdims=True)

    @pl.when(p == 1)
    def _():
        hblk = hv[pl.ds(j * r, r), :]
        h_ref[...] = hblk
        hb = (hblk + s_ref[...] * (1.0 / N)) * 0.5
        eh = jnp.dot(hb, wh_ref[...], preferred_element_type=F32) + bh_ref[...]
        et = jnp.dot(hb, wt_ref[...], preferred_element_type=F32) + bt_ref[...]
        eh_ref[...] = eh
        et_ref[...] = et
        kv = (jnp.dot(et, wkv_ref[...], preferred_element_type=F32)
              + bkv_ref[...])
        kk_ref[...] = kv[:, :D]
        vv_ref[...] = kv[:, D:].astype(jnp.bfloat16)


def _k3_body(eh_ref, et_ref, idx_ref, cnt_ref):
    logits = lax.dot_general(eh_ref[...] * SCALE, et_ref[...],
                             (((1,), (1,)), ((), ())),
                             preferred_element_type=F32)
    cols = lax.broadcasted_iota(I32, logits.shape, 1)
    l = logits
    outs = []
    for _ in range(T):
        m = jnp.max(l, axis=1, keepdims=True)
        cand = jnp.where(l == m, cols, I32(N))
        j = jnp.min(cand, axis=1, keepdims=True)
        outs.append(j)
        l = jnp.where(cols == j, -jnp.inf, l)
    idx_ref[...] = jnp.concatenate(outs, axis=1)
    cnt_ref[...] = jnp.sum((l == -jnp.inf).astype(I32), axis=0,
                           keepdims=True).reshape(cnt_ref.shape)


def _sc_gather(v, idx_flat):
    info = plsc.get_sparse_core_info()
    nc = info.num_cores
    nw = nc * info.num_subcores
    b_per_w = (N * T) // nw
    ch = 96
    n_ch = b_per_w // ch
    dw = D // 2
    mesh = plsc.VectorSubcoreMesh(core_axis_name="c", subcore_axis_name="s")

    @functools.partial(
        pl.kernel,
        out_type=jax.ShapeDtypeStruct((N * T, dw), I32),
        mesh=mesh,
        scratch_types=[
            pltpu.VMEM((b_per_w,), I32),
            pltpu.VMEM((2, ch, dw), I32),
            pltpu.SemaphoreType.DMA((2,)),
            pltpu.SemaphoreType.DMA((2,)),
        ],
    )
    def sc_k(v_hbm, idx_hbm, gv_hbm, idx_v, vbuf, vsem, vosem):
        wid = lax.axis_index("s") * nc + lax.axis_index("c")
        base = wid * b_per_w
        pltpu.sync_copy(idx_hbm.at[pl.ds(base, b_per_w)], idx_v)
        gath = [None, None]
        stor = [None, None]
        for c in range(n_ch):
            b = c % 2
            if stor[b] is not None:
                stor[b].wait()
                stor[b] = None
            idxc = idx_v.at[pl.ds(c * ch, ch)]
            gath[b] = (
                pltpu.async_copy(v_hbm.at[idxc], vbuf.at[b], vsem.at[b]),
                base + c * ch,
            )
            pb = (c - 1) % 2
            if c > 0 and gath[pb] is not None:
                gv_d, poff = gath[pb]
                gv_d.wait()
                gath[pb] = None
                stor[pb] = pltpu.async_copy(
                    vbuf.at[pb], gv_hbm.at[pl.ds(poff, ch)], vosem.at[pb])
        lb = (n_ch - 1) % 2
        gv_d, poff = gath[lb]
        gv_d.wait()
        stor[lb] = pltpu.async_copy(
            vbuf.at[lb], gv_hbm.at[pl.ds(poff, ch)], vosem.at[lb])
        for b in range(2):
            if stor[b] is not None:
                stor[b].wait()

    return sc_k(v, idx_flat)


def _k4_body(c_ref, s_ref, h_ref, x_ref, wq_ref, bq_ref, kk_ref,
             s16_ref, ins_ref):
    counts = jnp.sum(c_ref[...], axis=0, keepdims=True)
    m = jnp.max(counts)
    lane = lax.broadcasted_iota(I32, counts.shape, 1)
    top = jnp.min(jnp.where(counts == m, lane, I32(N)))
    onehot = (lane == top).astype(F32)
    htop = jnp.dot(onehot, h_ref[...], preferred_element_type=F32)
    ins_ref[...] = jnp.dot(onehot, x_ref[...], preferred_element_type=F32)
    qrow = (htop + s_ref[...] * (1.0 / N)) * 0.5
    q = (jnp.dot(qrow, wq_ref[...], preferred_element_type=F32)
         + bq_ref[...]) * SCALE
    kq = kk_ref[...] * q
    parts = [jnp.sum(kq[:, hh * HD:(hh + 1) * HD], axis=1, keepdims=True)
             for hh in range(NH)]
    parts.append(jnp.zeros((N, 16 - NH), F32))
    s16_ref[...] = jnp.concatenate(parts, axis=1)


def _layernorm(v, gamma, beta):
    mu = jnp.mean(v, axis=-1, keepdims=True)
    var = jnp.mean((v - mu) ** 2, axis=-1, keepdims=True)
    return (v - mu) / jnp.sqrt(var + 1e-5) * gamma + beta


def _k6_body(gv_ref, idx_ref, s16_ref, eh_ref, x_ref, wl_ref, bl_ref,
             gm_ref, bt_ref, out_ref):
    vpart = gv_ref[...].astype(F32)
    idxb = idx_ref[...]
    r = idxb.shape[0]
    col = lax.broadcasted_iota(I32, (r, N), 1)
    wls = []
    for t in range(T):
        onehot = (col == idxb[:, t:t + 1]).astype(F32)
        wls.append(jnp.dot(onehot, s16_ref[...],
                           preferred_element_type=F32))
    m = wls[0]
    for t in range(1, T):
        m = jnp.maximum(m, wls[t])
    es = [jnp.exp(w - m) for w in wls]
    den = es[0]
    for t in range(1, T):
        den = den + es[t]
    ws = [e / den for e in es]
    nb_parts = []
    for h in range(NH):
        acc = None
        for t in range(T):
            contrib = ws[t][:, h:h + 1] * vpart[:, t, h * HD:(h + 1) * HD]
            acc = contrib if acc is None else acc + contrib
        nb_parts.append(acc)
    nb = jnp.concatenate(nb_parts, axis=1)
    gamma = gm_ref[...]
    beta = bt_ref[...]
    nb = _layernorm(nb, gamma, beta)
    t2 = _layernorm(eh_ref[...] + nb, gamma, beta)
    o = jnp.dot(t2, wl_ref[...], preferred_element_type=F32) + bl_ref[...]
    o = jnp.where(o >= 0, o, 0.01 * o)
    out_ref[...] = o + x_ref[...]


def _full(shape):
    nd = len(shape)
    return pl.BlockSpec(shape, lambda i: (0,) * nd)


def kernel(x, W1, b1, Wh, bh, Wt, bt, Wq, bq, Wkv, bkv, Wl, bl, gamma, beta):
    x2 = x.reshape(N, D)
    b1r = b1.reshape(1, D)
    bhr = bh.reshape(1, D)
    btr = bt.reshape(1, D)
    bqr = bq.reshape(1, D)
    bkvr = bkv.reshape(1, KVD)
    blr = bl.reshape(1, D)
    gmr = gamma.reshape(1, D)
    ber = beta.reshape(1, D)

    r2 = 512
    blk = pl.BlockSpec((r2, D), lambda p, j: (j, 0))
    f2 = lambda shp: pl.BlockSpec(shp, lambda p, j: (0,) * len(shp))
    h, s, eh, et, kk, vv = pl.pallas_call(
        _k12_body,
        grid=(2, N // r2),
        in_specs=[blk,
                  f2((D, D)), f2((1, D)),
                  f2((D, D)), f2((1, D)),
                  f2((D, D)), f2((1, D)),
                  f2((D, KVD)), f2((1, KVD))],
        out_specs=[blk, f2((1, D)), blk, blk, blk,
                   pl.BlockSpec((r2, D), lambda p, j: (j, 0))],
        out_shape=[jax.ShapeDtypeStruct((N, D), F32),
                   jax.ShapeDtypeStruct((1, D), F32),
                   jax.ShapeDtypeStruct((N, D), F32),
                   jax.ShapeDtypeStruct((N, D), F32),
                   jax.ShapeDtypeStruct((N, D), F32),
                   jax.ShapeDtypeStruct((N, D), jnp.bfloat16)],
        scratch_shapes=[pltpu.VMEM((N, D), F32)],
    )(x2, W1, b1r, Wh, bhr, Wt, btr, Wkv, bkvr)

    r3 = 256
    idx8, cnts = pl.pallas_call(
        _k3_body,
        grid=(N // r3,),
        in_specs=[pl.BlockSpec((r3, D), lambda i: (i, 0)), _full((N, D))],
        out_specs=[pl.BlockSpec((r3, T), lambda i: (i, 0)),
                   pl.BlockSpec((1, 1, N), lambda i: (i, 0, 0))],
        out_shape=[jax.ShapeDtypeStruct((N, T), I32),
                   jax.ShapeDtypeStruct((N // r3, 1, N), I32)],
    )(eh, et)

    cnts2 = cnts.reshape(N // r3, N)
    s16, ins = pl.pallas_call(
        _k4_body,
        grid=(1,),
        in_specs=[_full(cnts2.shape), _full((1, D)), _full((N, D)),
                  _full((N, D)), _full((D, D)), _full((1, D)),
                  _full((N, D))],
        out_specs=[_full((N, 16)), _full((1, D))],
        out_shape=[jax.ShapeDtypeStruct((N, 16), F32),
                   jax.ShapeDtypeStruct((1, D), F32)],
    )(cnts2, s, h, x2, Wq, bqr, kk)

    idx_flat = idx8.reshape(N * T)
    vvp = lax.bitcast_convert_type(vv.reshape(N, D // 2, 2), I32)
    gv = _sc_gather(vvp, idx_flat)

    r6 = 128
    gv3 = lax.bitcast_convert_type(gv, jnp.bfloat16).reshape(N, T, D)
    emb = pl.pallas_call(
        _k6_body,
        grid=(N // r6,),
        in_specs=[pl.BlockSpec((r6, T, D), lambda i: (i, 0, 0)),
                  pl.BlockSpec((r6, T), lambda i: (i, 0)),
                  _full((N, 16)),
                  pl.BlockSpec((r6, D), lambda i: (i, 0)),
                  pl.BlockSpec((r6, D), lambda i: (i, 0)),
                  _full((D, D)), _full((1, D)), _full((1, D)), _full((1, D))],
        out_specs=pl.BlockSpec((r6, D), lambda i: (i, 0)),
        out_shape=jax.ShapeDtypeStruct((N, D), F32),
    )(gv3, idx8, s16, eh, x2, Wl, blr, gmr, ber)

    return emb.reshape(1, N, D), ins.reshape(1, 1, D)


# speedup vs baseline: 1.0316x; 1.0316x over previous
import functools

import jax
import jax.numpy as jnp
from jax import lax
from jax.experimental import pallas as pl
from jax.experimental.pallas import tpu as pltpu
from jax.experimental.pallas import tpu_sc as plsc

N = 4096
D = 512
KVD = 1024
T = 6
TP = 8
NH = 8
HD = 64
SCALE = float(HD) ** -0.5
SW = 128
F32 = jnp.float32
I32 = jnp.int32


def _k12_body(x_ref, w1_ref, b1_ref, wh_ref, bh_ref, wt_ref, bt_ref,
              wkv_ref, bkv_ref,
              h_ref, s_ref, eh_ref, et_ref, kk_ref, vv_ref, hv):
    p = pl.program_id(0)
    j = pl.program_id(1)
    r = x_ref.shape[0]

    @pl.when(p == 0)
    def _():
        h = (jnp.dot(x_ref[...], w1_ref[...], preferred_element_type=F32)
             + b1_ref[...])
        h = jnp.where(h >= 0, h, 0.01 * h)
        h_ref[...] = h
        hv[pl.ds(j * r, r), :] = h

        @pl.when(j == 0)
        def _():
            s_ref[...] = jnp.zeros_like(s_ref)

        s_ref[...] += jnp.sum(h, axis=0, keepdims=True)

    @pl.when(p == 1)
    def _():
        hblk = hv[pl.ds(j * r, r), :]
        h_ref[...] = hblk
        hb = (hblk + s_ref[...] * (1.0 / N)) * 0.5
        eh = jnp.dot(hb, wh_ref[...], preferred_element_type=F32) + bh_ref[...]
        et = jnp.dot(hb, wt_ref[...], preferred_element_type=F32) + bt_ref[...]
        eh_ref[...] = eh
        et_ref[...] = et
        kv = (jnp.dot(et, wkv_ref[...], preferred_element_type=F32)
              + bkv_ref[...])
        kk_ref[...] = kv[:, :D]
        vv_ref[...] = kv[:, D:].astype(jnp.bfloat16)


def _k3_body(eh_ref, et_ref, idx_ref, cnt_ref):
    logits = lax.dot_general(eh_ref[...] * SCALE, et_ref[...],
                             (((1,), (1,)), ((), ())),
                             preferred_element_type=F32)
    cols = lax.broadcasted_iota(I32, logits.shape, 1)
    l = logits
    outs = []
    for _ in range(T):
        m = jnp.max(l, axis=1, keepdims=True)
        cand = jnp.where(l == m, cols, I32(N))
        j = jnp.min(cand, axis=1, keepdims=True)
        outs.append(j)
        l = jnp.where(cols == j, -jnp.inf, l)
    idx_ref[...] = jnp.concatenate(outs, axis=1)
    cnt_ref[...] = jnp.sum((l == -jnp.inf).astype(I32), axis=0,
                           keepdims=True).reshape(cnt_ref.shape)


def _sc_gather(v, s16, idx_flat):
    info = plsc.get_sparse_core_info()
    nc = info.num_cores
    nw = nc * info.num_subcores
    b_per_w = (N * T) // nw
    ch = 96
    n_ch = b_per_w // ch
    dw = D // 2
    mesh = plsc.VectorSubcoreMesh(core_axis_name="c", subcore_axis_name="s")

    @functools.partial(
        pl.kernel,
        out_type=(jax.ShapeDtypeStruct((N * T, dw), I32),
                  jax.ShapeDtypeStruct((N * T, SW), F32)),
        mesh=mesh,
        scratch_types=[
            pltpu.VMEM((b_per_w,), I32),
            pltpu.VMEM((2, ch, dw), I32),
            pltpu.VMEM((2, ch, SW), F32),
            pltpu.SemaphoreType.DMA((2,)),
            pltpu.SemaphoreType.DMA((2,)),
            pltpu.SemaphoreType.DMA((2,)),
            pltpu.SemaphoreType.DMA((2,)),
        ],
    )
    def sc_k(v_hbm, s_hbm, idx_hbm, gv_hbm, gs_hbm, idx_v, vbuf, sbuf,
             vsem, ssem, vosem, sosem):
        wid = lax.axis_index("s") * nc + lax.axis_index("c")
        base = wid * b_per_w
        pltpu.sync_copy(idx_hbm.at[pl.ds(base, b_per_w)], idx_v)
        gath = [None, None]
        stor = [None, None]
        for c in range(n_ch):
            b = c % 2
            if stor[b] is not None:
                for d in stor[b]:
                    d.wait()
                stor[b] = None
            idxc = idx_v.at[pl.ds(c * ch, ch)]
            gath[b] = (
                pltpu.async_copy(v_hbm.at[idxc], vbuf.at[b], vsem.at[b]),
                pltpu.async_copy(s_hbm.at[idxc], sbuf.at[b], ssem.at[b]),
                base + c * ch,
            )
            pb = (c - 1) % 2
            if c > 0 and gath[pb] is not None:
                gv_d, gs_d, poff = gath[pb]
                gv_d.wait()
                gs_d.wait()
                gath[pb] = None
                stor[pb] = (
                    pltpu.async_copy(vbuf.at[pb], gv_hbm.at[pl.ds(poff, ch)],
                                     vosem.at[pb]),
                    pltpu.async_copy(sbuf.at[pb], gs_hbm.at[pl.ds(poff, ch)],
                                     sosem.at[pb]),
                )
        lb = (n_ch - 1) % 2
        gv_d, gs_d, poff = gath[lb]
        gv_d.wait()
        gs_d.wait()
        stor[lb] = (
            pltpu.async_copy(vbuf.at[lb], gv_hbm.at[pl.ds(poff, ch)],
                             vosem.at[lb]),
            pltpu.async_copy(sbuf.at[lb], gs_hbm.at[pl.ds(poff, ch)],
                             sosem.at[lb]),
        )
        for b in range(2):
            if stor[b] is not None:
                for d in stor[b]:
                    d.wait()

    return sc_k(v, s16, idx_flat)


def _k4_body(c_ref, s_ref, h_ref, x_ref, wq_ref, bq_ref, kk_ref,
             s16_ref, ins_ref):
    counts = jnp.sum(c_ref[...], axis=0, keepdims=True)
    m = jnp.max(counts)
    lane = lax.broadcasted_iota(I32, counts.shape, 1)
    top = jnp.min(jnp.where(counts == m, lane, I32(N)))
    onehot = (lane == top).astype(F32)
    htop = jnp.dot(onehot, h_ref[...], preferred_element_type=F32)
    ins_ref[...] = jnp.dot(onehot, x_ref[...], preferred_element_type=F32)
    qrow = (htop + s_ref[...] * (1.0 / N)) * 0.5
    q = (jnp.dot(qrow, wq_ref[...], preferred_element_type=F32)
         + bq_ref[...]) * SCALE
    kq = kk_ref[...] * q
    parts = [jnp.sum(kq[:, hh * HD:(hh + 1) * HD], axis=1, keepdims=True)
             for hh in range(NH)]
    parts.append(jnp.zeros((N, SW - NH), F32))
    s16_ref[...] = jnp.concatenate(parts, axis=1)


def _layernorm(v, gamma, beta):
    mu = jnp.mean(v, axis=-1, keepdims=True)
    var = jnp.mean((v - mu) ** 2, axis=-1, keepdims=True)
    return (v - mu) / jnp.sqrt(var + 1e-5) * gamma + beta


def _k6_body(gv_ref, gs_ref, eh_ref, x_ref, wl_ref, bl_ref, gm_ref,
             bt_ref, out_ref):
    vpart = gv_ref[...].astype(F32)
    wl = gs_ref[...][:, :, :NH]
    m = jnp.max(wl, axis=1, keepdims=True)
    e = jnp.exp(wl - m)
    w = e / jnp.sum(e, axis=1, keepdims=True)
    nb_parts = []
    for h in range(NH):
        wh = w[:, :, h:h + 1]
        vh = vpart[:, :, h * HD:(h + 1) * HD]
        nb_parts.append(jnp.sum(wh * vh, axis=1))
    nb = jnp.concatenate(nb_parts, axis=1)
    gamma = gm_ref[...]
    beta = bt_ref[...]
    nb = _layernorm(nb, gamma, beta)
    t2 = _layernorm(eh_ref[...] + nb, gamma, beta)
    o = jnp.dot(t2, wl_ref[...], preferred_element_type=F32) + bl_ref[...]
    o = jnp.where(o >= 0, o, 0.01 * o)
    out_ref[...] = o + x_ref[...]


def _full(shape):
    nd = len(shape)
    return pl.BlockSpec(shape, lambda i: (0,) * nd)


def kernel(x, W1, b1, Wh, bh, Wt, bt, Wq, bq, Wkv, bkv, Wl, bl, gamma, beta):
    x2 = x.reshape(N, D)
    b1r = b1.reshape(1, D)
    bhr = bh.reshape(1, D)
    btr = bt.reshape(1, D)
    bqr = bq.reshape(1, D)
    bkvr = bkv.reshape(1, KVD)
    blr = bl.reshape(1, D)
    gmr = gamma.reshape(1, D)
    ber = beta.reshape(1, D)

    r2 = 512
    blk = pl.BlockSpec((r2, D), lambda p, j: (j, 0))
    f2 = lambda shp: pl.BlockSpec(shp, lambda p, j: (0,) * len(shp))
    h, s, eh, et, kk, vv = pl.pallas_call(
        _k12_body,
        grid=(2, N // r2),
        in_specs=[blk,
                  f2((D, D)), f2((1, D)),
                  f2((D, D)), f2((1, D)),
                  f2((D, D)), f2((1, D)),
                  f2((D, KVD)), f2((1, KVD))],
        out_specs=[blk, f2((1, D)), blk, blk, blk,
                   pl.BlockSpec((r2, D), lambda p, j: (j, 0))],
        out_shape=[jax.ShapeDtypeStruct((N, D), F32),
                   jax.ShapeDtypeStruct((1, D), F32),
                   jax.ShapeDtypeStruct((N, D), F32),
                   jax.ShapeDtypeStruct((N, D), F32),
                   jax.ShapeDtypeStruct((N, D), F32),
                   jax.ShapeDtypeStruct((N, D), jnp.bfloat16)],
        scratch_shapes=[pltpu.VMEM((N, D), F32)],
    )(x2, W1, b1r, Wh, bhr, Wt, btr, Wkv, bkvr)

    r3 = 256
    idx8, cnts = pl.pallas_call(
        _k3_body,
        grid=(N // r3,),
        in_specs=[pl.BlockSpec((r3, D), lambda i: (i, 0)), _full((N, D))],
        out_specs=[pl.BlockSpec((r3, T), lambda i: (i, 0)),
                   pl.BlockSpec((1, 1, N), lambda i: (i, 0, 0))],
        out_shape=[jax.ShapeDtypeStruct((N, T), I32),
                   jax.ShapeDtypeStruct((N // r3, 1, N), I32)],
    )(eh, et)

    cnts2 = cnts.reshape(N // r3, N)
    s16, ins = pl.pallas_call(
        _k4_body,
        grid=(1,),
        in_specs=[_full(cnts2.shape), _full((1, D)), _full((N, D)),
                  _full((N, D)), _full((D, D)), _full((1, D)),
                  _full((N, D))],
        out_specs=[_full((N, SW)), _full((1, D))],
        out_shape=[jax.ShapeDtypeStruct((N, SW), F32),
                   jax.ShapeDtypeStruct((1, D), F32)],
    )(cnts2, s, h, x2, Wq, bqr, kk)

    idx_flat = idx8.reshape(N * T)
    vvp = lax.bitcast_convert_type(vv.reshape(N, D // 2, 2), I32)
    gv, gs = _sc_gather(vvp, s16, idx_flat)

    r6 = 128
    gv3 = lax.bitcast_convert_type(gv, jnp.bfloat16).reshape(N, T, D)
    gs3 = gs.reshape(N, T, SW)
    emb = pl.pallas_call(
        _k6_body,
        grid=(N // r6,),
        in_specs=[pl.BlockSpec((r6, T, D), lambda i: (i, 0, 0)),
                  pl.BlockSpec((r6, T, SW), lambda i: (i, 0, 0)),
                  pl.BlockSpec((r6, D), lambda i: (i, 0)),
                  pl.BlockSpec((r6, D), lambda i: (i, 0)),
                  _full((D, D)), _full((1, D)), _full((1, D)), _full((1, D))],
        out_specs=pl.BlockSpec((r6, D), lambda i: (i, 0)),
        out_shape=jax.ShapeDtypeStruct((N, D), F32),
    )(gv3, gs3, eh, x2, Wl, blr, gmr, ber)

    return emb.reshape(1, N, D), ins.reshape(1, 1, D)


# speedup vs baseline: 4.5552x; 4.4158x over previous
import functools

import jax
import jax.numpy as jnp
from jax import lax
from jax.experimental import pallas as pl
from jax.experimental.pallas import tpu as pltpu
from jax.experimental.pallas import tpu_sc as plsc

N = 4096
D = 512
KVD = 1024
T = 6
TP = 8
NH = 8
HD = 64
SCALE = float(HD) ** -0.5
SW = 128
F32 = jnp.float32
I32 = jnp.int32


def _k12_body(x_ref, w1_ref, b1_ref, wh_ref, bh_ref, wt_ref, bt_ref,
              wkv_ref, bkv_ref,
              h_ref, s_ref, eh_ref, et_ref, kk_ref, vv_ref, hv):
    p = pl.program_id(0)
    j = pl.program_id(1)
    r = x_ref.shape[0]

    @pl.when(p == 0)
    def _():
        h = (jnp.dot(x_ref[...], w1_ref[...], preferred_element_type=F32)
             + b1_ref[...])
        h = jnp.where(h >= 0, h, 0.01 * h)
        h_ref[...] = h
        hv[pl.ds(j * r, r), :] = h

        @pl.when(j == 0)
        def _():
            s_ref[...] = jnp.zeros_like(s_ref)

        s_ref[...] += jnp.sum(h, axis=0, keepdims=True)

    @pl.when(p == 1)
    def _():
        hblk = hv[pl.ds(j * r, r), :]
        h_ref[...] = hblk
        hb = (hblk + s_ref[...] * (1.0 / N)) * 0.5
        eh = jnp.dot(hb, wh_ref[...], preferred_element_type=F32) + bh_ref[...]
        et = jnp.dot(hb, wt_ref[...], preferred_element_type=F32) + bt_ref[...]
        eh_ref[...] = eh
        et_ref[...] = et
        kv = (jnp.dot(et, wkv_ref[...], preferred_element_type=F32)
              + bkv_ref[...])
        kk_ref[...] = kv[:, :D]
        v1 = lax.bitcast_convert_type(
            kv[:, D:D + D // 2].astype(jnp.bfloat16).astype(F32), I32)
        v2 = lax.bitcast_convert_type(
            kv[:, D + D // 2:].astype(jnp.bfloat16).astype(F32), I32)
        vv_ref[...] = (v1 & I32(-65536)) | lax.shift_right_logical(v2, 16)


def _k3_body(eh_ref, et_ref, idx_ref, cnt_ref):
    logits = lax.dot_general(eh_ref[...] * SCALE, et_ref[...],
                             (((1,), (1,)), ((), ())),
                             preferred_element_type=F32)
    cols = lax.broadcasted_iota(I32, logits.shape, 1)
    l = logits
    outs = []
    for _ in range(T):
        m = jnp.max(l, axis=1, keepdims=True)
        cand = jnp.where(l == m, cols, I32(N))
        j = jnp.min(cand, axis=1, keepdims=True)
        outs.append(j)
        l = jnp.where(cols == j, -jnp.inf, l)
    idx_ref[...] = jnp.concatenate(outs, axis=1)
    cnt_ref[...] = jnp.sum((l == -jnp.inf).astype(I32), axis=0,
                           keepdims=True).reshape(cnt_ref.shape)


def _sc_gather(v, s16, idx_flat):
    info = plsc.get_sparse_core_info()
    nc = info.num_cores
    nw = nc * info.num_subcores
    b_per_w = (N * T) // nw
    ch = 96
    n_ch = b_per_w // ch
    dw = D // 2
    mesh = plsc.VectorSubcoreMesh(core_axis_name="c", subcore_axis_name="s")

    @functools.partial(
        pl.kernel,
        out_type=(jax.ShapeDtypeStruct((N * T, dw), I32),
                  jax.ShapeDtypeStruct((N * T, SW), F32)),
        mesh=mesh,
        scratch_types=[
            pltpu.VMEM((b_per_w,), I32),
            pltpu.VMEM((2, ch, dw), I32),
            pltpu.VMEM((2, ch, SW), F32),
            pltpu.SemaphoreType.DMA((2,)),
            pltpu.SemaphoreType.DMA((2,)),
            pltpu.SemaphoreType.DMA((2,)),
            pltpu.SemaphoreType.DMA((2,)),
        ],
    )
    def sc_k(v_hbm, s_hbm, idx_hbm, gv_hbm, gs_hbm, idx_v, vbuf, sbuf,
             vsem, ssem, vosem, sosem):
        wid = lax.axis_index("s") * nc + lax.axis_index("c")
        base = wid * b_per_w
        pltpu.sync_copy(idx_hbm.at[pl.ds(base, b_per_w)], idx_v)
        gath = [None, None]
        stor = [None, None]
        for c in range(n_ch):
            b = c % 2
            if stor[b] is not None:
                for d in stor[b]:
                    d.wait()
                stor[b] = None
            idxc = idx_v.at[pl.ds(c * ch, ch)]
            gath[b] = (
                pltpu.async_copy(v_hbm.at[idxc], vbuf.at[b], vsem.at[b]),
                pltpu.async_copy(s_hbm.at[idxc], sbuf.at[b], ssem.at[b]),
                base + c * ch,
            )
            pb = (c - 1) % 2
            if c > 0 and gath[pb] is not None:
                gv_d, gs_d, poff = gath[pb]
                gv_d.wait()
                gs_d.wait()
                gath[pb] = None
                stor[pb] = (
                    pltpu.async_copy(vbuf.at[pb], gv_hbm.at[pl.ds(poff, ch)],
                                     vosem.at[pb]),
                    pltpu.async_copy(sbuf.at[pb], gs_hbm.at[pl.ds(poff, ch)],
                                     sosem.at[pb]),
                )
        lb = (n_ch - 1) % 2
        gv_d, gs_d, poff = gath[lb]
        gv_d.wait()
        gs_d.wait()
        stor[lb] = (
            pltpu.async_copy(vbuf.at[lb], gv_hbm.at[pl.ds(poff, ch)],
                             vosem.at[lb]),
            pltpu.async_copy(sbuf.at[lb], gs_hbm.at[pl.ds(poff, ch)],
                             sosem.at[lb]),
        )
        for b in range(2):
            if stor[b] is not None:
                for d in stor[b]:
                    d.wait()

    return sc_k(v, s16, idx_flat)


def _k4_body(c_ref, s_ref, h_ref, x_ref, wq_ref, bq_ref, kk_ref,
             s16_ref, ins_ref):
    counts = jnp.sum(c_ref[...], axis=0, keepdims=True)
    m = jnp.max(counts)
    lane = lax.broadcasted_iota(I32, counts.shape, 1)
    top = jnp.min(jnp.where(counts == m, lane, I32(N)))
    onehot = (lane == top).astype(F32)
    htop = jnp.dot(onehot, h_ref[...], preferred_element_type=F32)
    ins_ref[...] = jnp.dot(onehot, x_ref[...], preferred_element_type=F32)
    qrow = (htop + s_ref[...] * (1.0 / N)) * 0.5
    q = (jnp.dot(qrow, wq_ref[...], preferred_element_type=F32)
         + bq_ref[...]) * SCALE
    kq = kk_ref[...] * q
    parts = [jnp.sum(kq[:, hh * HD:(hh + 1) * HD], axis=1, keepdims=True)
             for hh in range(NH)]
    parts.append(jnp.zeros((N, SW - NH), F32))
    s16_ref[...] = jnp.concatenate(parts, axis=1)


def _layernorm(v, gamma, beta):
    mu = jnp.mean(v, axis=-1, keepdims=True)
    var = jnp.mean((v - mu) ** 2, axis=-1, keepdims=True)
    return (v - mu) / jnp.sqrt(var + 1e-5) * gamma + beta


def _k6_body(gv_ref, gs_ref, eh_ref, x_ref, wl_ref, bl_ref, gm_ref,
             bt_ref, out_ref):
    pv = gv_ref[...]
    v1 = lax.bitcast_convert_type(pv & I32(-65536), F32)
    v2 = lax.bitcast_convert_type(lax.shift_left(pv, 16), F32)
    wl = gs_ref[...][:, :, :NH]
    m = jnp.max(wl, axis=1, keepdims=True)
    e = jnp.exp(wl - m)
    w = e / jnp.sum(e, axis=1, keepdims=True)
    nb_parts = []
    hw = D // 2
    for h in range(NH):
        wh = w[:, :, h:h + 1]
        src_half = v1 if h < NH // 2 else v2
        off = h * HD if h < NH // 2 else h * HD - hw
        vh = src_half[:, :, off:off + HD]
        nb_parts.append(jnp.sum(wh * vh, axis=1))
    nb = jnp.concatenate(nb_parts, axis=1)
    gamma = gm_ref[...]
    beta = bt_ref[...]
    nb = _layernorm(nb, gamma, beta)
    t2 = _layernorm(eh_ref[...] + nb, gamma, beta)
    o = jnp.dot(t2, wl_ref[...], preferred_element_type=F32) + bl_ref[...]
    o = jnp.where(o >= 0, o, 0.01 * o)
    out_ref[...] = o + x_ref[...]


def _full(shape):
    nd = len(shape)
    return pl.BlockSpec(shape, lambda i: (0,) * nd)


def kernel(x, W1, b1, Wh, bh, Wt, bt, Wq, bq, Wkv, bkv, Wl, bl, gamma, beta):
    x2 = x.reshape(N, D)
    b1r = b1.reshape(1, D)
    bhr = bh.reshape(1, D)
    btr = bt.reshape(1, D)
    bqr = bq.reshape(1, D)
    bkvr = bkv.reshape(1, KVD)
    blr = bl.reshape(1, D)
    gmr = gamma.reshape(1, D)
    ber = beta.reshape(1, D)

    r2 = 512
    blk = pl.BlockSpec((r2, D), lambda p, j: (j, 0))
    f2 = lambda shp: pl.BlockSpec(shp, lambda p, j: (0,) * len(shp))
    h, s, eh, et, kk, vv = pl.pallas_call(
        _k12_body,
        grid=(2, N // r2),
        in_specs=[blk,
                  f2((D, D)), f2((1, D)),
                  f2((D, D)), f2((1, D)),
                  f2((D, D)), f2((1, D)),
                  f2((D, KVD)), f2((1, KVD))],
        out_specs=[blk, f2((1, D)), blk, blk, blk,
                   pl.BlockSpec((r2, D // 2), lambda p, j: (j, 0))],
        out_shape=[jax.ShapeDtypeStruct((N, D), F32),
                   jax.ShapeDtypeStruct((1, D), F32),
                   jax.ShapeDtypeStruct((N, D), F32),
                   jax.ShapeDtypeStruct((N, D), F32),
                   jax.ShapeDtypeStruct((N, D), F32),
                   jax.ShapeDtypeStruct((N, D // 2), I32)],
        scratch_shapes=[pltpu.VMEM((N, D), F32)],
    )(x2, W1, b1r, Wh, bhr, Wt, btr, Wkv, bkvr)

    r3 = 256
    idx8, cnts = pl.pallas_call(
        _k3_body,
        grid=(N // r3,),
        in_specs=[pl.BlockSpec((r3, D), lambda i: (i, 0)), _full((N, D))],
        out_specs=[pl.BlockSpec((r3, T), lambda i: (i, 0)),
                   pl.BlockSpec((1, 1, N), lambda i: (i, 0, 0))],
        out_shape=[jax.ShapeDtypeStruct((N, T), I32),
                   jax.ShapeDtypeStruct((N // r3, 1, N), I32)],
    )(eh, et)

    cnts2 = cnts.reshape(N // r3, N)
    s16, ins = pl.pallas_call(
        _k4_body,
        grid=(1,),
        in_specs=[_full(cnts2.shape), _full((1, D)), _full((N, D)),
                  _full((N, D)), _full((D, D)), _full((1, D)),
                  _full((N, D))],
        out_specs=[_full((N, SW)), _full((1, D))],
        out_shape=[jax.ShapeDtypeStruct((N, SW), F32),
                   jax.ShapeDtypeStruct((1, D), F32)],
    )(cnts2, s, h, x2, Wq, bqr, kk)

    idx_flat = idx8.reshape(N * T)
    gv, gs = _sc_gather(vv, s16, idx_flat)

    r6 = 128
    gv3 = gv.reshape(N, T, D // 2)
    gs3 = gs.reshape(N, T, SW)
    emb = pl.pallas_call(
        _k6_body,
        grid=(N // r6,),
        in_specs=[pl.BlockSpec((r6, T, D // 2), lambda i: (i, 0, 0)),
                  pl.BlockSpec((r6, T, SW), lambda i: (i, 0, 0)),
                  pl.BlockSpec((r6, D), lambda i: (i, 0)),
                  pl.BlockSpec((r6, D), lambda i: (i, 0)),
                  _full((D, D)), _full((1, D)), _full((1, D)), _full((1, D))],
        out_specs=pl.BlockSpec((r6, D), lambda i: (i, 0)),
        out_shape=jax.ShapeDtypeStruct((N, D), F32),
    )(gv3, gs3, eh, x2, Wl, blr, gmr, ber)

    return emb.reshape(1, N, D), ins.reshape(1, 1, D)
